# Initial kernel scaffold; baseline (speedup 1.0000x reference)
import jax
import jax.numpy as jnp
from jax.experimental import pallas as pl


def kernel(x, mean, stddev, outputs):
    raise NotImplementedError("write your pallas kernel here")


# speedup vs baseline: 19.5563x; 19.5563x over previous
import functools

import jax
import jax.numpy as jnp
from jax.experimental import pallas as pl
from jax.experimental.pallas import tpu as pltpu

B = 1024
K = 1000
D = 128
OUT = 64
TOP_K = 10
KP = 1024
BB = 256

_NEG = -3.0e38


def _body(x_ref, mean_ref, stddev_ref, outputs_ref, out_ref):
    x = x_ref[...]
    mean = mean_ref[...]
    std = stddev_ref[...]
    outs = outputs_ref[...]

    iv = 1.0 / (std * std)
    w2 = mean * iv
    bias = (jnp.sum(-0.5 * mean * w2 - jnp.log(std), axis=1)
            - 0.5 * D * jnp.log(2.0 * jnp.pi))

    t1 = jax.lax.dot_general(x * x, iv, (((1,), (1,)), ((), ())),
                             preferred_element_type=jnp.float32,
                             precision=jax.lax.Precision.HIGHEST)
    t2 = jax.lax.dot_general(x, w2, (((1,), (1,)), ((), ())),
                             preferred_element_type=jnp.float32,
                             precision=jax.lax.Precision.HIGHEST)
    lp = -0.5 * t1 + t2 + bias[None, :]
    lp = jnp.concatenate(
        [lp, jnp.full((BB, KP - K), _NEG, jnp.float32)], axis=1)

    iota = jax.lax.broadcasted_iota(jnp.int32, (BB, KP), 1)
    v = lp
    m0 = jnp.max(v, axis=1, keepdims=True)
    wmat = jnp.zeros((BB, KP), jnp.float32)
    wsum = jnp.zeros((BB, 1), jnp.float32)
    for _ in range(TOP_K):
        cur = jnp.max(v, axis=1, keepdims=True)
        ismax = v == cur
        first = jnp.min(jnp.where(ismax, iota, KP), axis=1, keepdims=True)
        onehot = iota == first
        e = jnp.exp(cur - m0)
        wmat = jnp.where(onehot, e, wmat)
        wsum = wsum + e
        v = jnp.where(onehot, _NEG, v)

    outs_p = jnp.concatenate(
        [outs, jnp.zeros((KP - K, OUT), jnp.float32)], axis=0)
    acc = jax.lax.dot_general(wmat, outs_p, (((1,), (0,)), ((), ())),
                              preferred_element_type=jnp.float32)
    out_ref[...] = acc / wsum


@jax.jit
def kernel(x, mean, stddev, outputs):
    grid = (B // BB,)
    return pl.pallas_call(
        _body,
        grid=grid,
        in_specs=[
            pl.BlockSpec((BB, D), lambda i: (i, 0)),
            pl.BlockSpec((K, D), lambda i: (0, 0)),
            pl.BlockSpec((K, D), lambda i: (0, 0)),
            pl.BlockSpec((K, OUT), lambda i: (0, 0)),
        ],
        out_specs=pl.BlockSpec((BB, OUT), lambda i: (i, 0)),
        out_shape=jax.ShapeDtypeStruct((B, OUT), jnp.float32),
        compiler_params=pltpu.CompilerParams(
            dimension_semantics=("arbitrary",)),
    )(x, mean, stddev, outputs)
